# baseline (speedup 1.0000x reference)
import functools

import jax
import jax.numpy as jnp
from jax import lax
from jax.experimental import pallas as pl
from jax.experimental.pallas import tpu as pltpu
from jax.experimental.pallas import tpu_sc as plsc

B = 16384
D = 128
NC = 2
NS = 16
NW = NC * NS
BPW = B // NW
CHUNK = 256
NCHUNK = BPW // CHUNK

_mesh = plsc.VectorSubcoreMesh(core_axis_name="c", subcore_axis_name="s")


@functools.partial(
    pl.kernel,
    mesh=_mesh,
    out_type=jax.ShapeDtypeStruct((D, NW, BPW), jnp.float32),
    compiler_params=pltpu.CompilerParams(needs_layout_passes=False),
    scratch_types=[
        pltpu.VMEM((BPW,), jnp.int32),
        pltpu.VMEM((CHUNK, D), jnp.float32),
        pltpu.VMEM((D, BPW), jnp.float32),
        pltpu.SemaphoreType.DMA,
    ],
)
def _sc_gather_t(table_hbm, idx_hbm, out_hbm, idx_v, rows_v, t_v, sem):
    wid = lax.axis_index("s") * NC + lax.axis_index("c")
    base = wid * BPW
    pltpu.sync_copy(idx_hbm.at[pl.ds(base, BPW)], idx_v)

    lane = lax.iota(jnp.int32, 16)

    def do_chunk(c, _):
        pltpu.async_copy(
            table_hbm.at[idx_v.at[pl.ds(c * CHUNK, CHUNK)]], rows_v, sem
        ).wait()

        def trow(i, _):
            col = jnp.full((16,), c * CHUNK + i, jnp.int32)
            for k in range(D // 16):
                v = rows_v[i, pl.ds(16 * k, 16)]
                plsc.store_scatter(t_v, [lane + 16 * k, col], v)
            return 0

        lax.fori_loop(0, CHUNK, trow, 0)
        return 0

    lax.fori_loop(0, NCHUNK, do_chunk, 0)
    pltpu.sync_copy(t_v, out_hbm.at[:, wid])


def kernel(task_index, task_index_to_mask):
    idx = task_index.reshape(B).astype(jnp.int32)
    out = _sc_gather_t(task_index_to_mask, idx)
    return out.reshape(D, B, 1)


# speedup vs baseline: 1.1351x; 1.1351x over previous
import functools

import jax
import jax.numpy as jnp
from jax import lax
from jax.experimental import pallas as pl
from jax.experimental.pallas import tpu as pltpu
from jax.experimental.pallas import tpu_sc as plsc

B = 16384
D = 128
NC = 2
NS = 16
NW = NC * NS
BPW = B // NW
CHUNK = 256
NCHUNK = BPW // CHUNK

_mesh = plsc.VectorSubcoreMesh(core_axis_name="c", subcore_axis_name="s")


@functools.partial(
    pl.kernel,
    mesh=_mesh,
    out_type=jax.ShapeDtypeStruct((D, NW, BPW), jnp.float32),
    compiler_params=pltpu.CompilerParams(needs_layout_passes=False),
    scratch_types=[
        pltpu.VMEM((BPW,), jnp.int32),
        pltpu.VMEM((CHUNK, D), jnp.float32),
        pltpu.VMEM((D, BPW), jnp.float32),
        pltpu.SemaphoreType.DMA,
    ],
)
def _sc_gather_t(table_hbm, idx_hbm, out_hbm, idx_v, rows_v, t_v, sem):
    wid = lax.axis_index("s") * NC + lax.axis_index("c")
    base = wid * BPW
    pltpu.sync_copy(idx_hbm.at[pl.ds(base, BPW)], idx_v)

    lane = lax.iota(jnp.int32, 16)

    for c in range(NCHUNK):
        pltpu.async_copy(
            table_hbm.at[idx_v.at[pl.ds(c * CHUNK, CHUNK)]], rows_v, sem
        ).wait()

        @plsc.parallel_loop(0, CHUNK, 1, unroll=4)
        def trow(i):
            col = jnp.full((16,), c * CHUNK + i, jnp.int32)
            for k in range(D // 16):
                v = rows_v[i, pl.ds(16 * k, 16)]
                plsc.store_scatter(t_v, [lane + 16 * k, col], v)

    pltpu.sync_copy(t_v, out_hbm.at[:, wid])


def kernel(task_index, task_index_to_mask):
    idx = task_index.reshape(B).astype(jnp.int32)
    out = _sc_gather_t(task_index_to_mask, idx)
    return out.reshape(D, B, 1)


# speedup vs baseline: 1.4312x; 1.2609x over previous
import functools

import jax
import jax.numpy as jnp
from jax import lax
from jax.experimental import pallas as pl
from jax.experimental.pallas import tpu as pltpu
from jax.experimental.pallas import tpu_sc as plsc

B = 16384
D = 128
NC = 2
NS = 16
NW = NC * NS
BPW = B // NW
CHUNK = 128
NCHUNK = BPW // CHUNK
RPITCH = D + 8

_mesh = plsc.VectorSubcoreMesh(core_axis_name="c", subcore_axis_name="s")


@functools.partial(
    pl.kernel,
    mesh=_mesh,
    out_type=jax.ShapeDtypeStruct((D, B), jnp.float32),
    compiler_params=pltpu.CompilerParams(needs_layout_passes=False),
    scratch_types=[
        pltpu.VMEM((BPW,), jnp.int32),
        pltpu.VMEM((CHUNK, RPITCH), jnp.float32),
        pltpu.VMEM((D, BPW), jnp.float32),
        pltpu.SemaphoreType.DMA,
    ],
)
def _sc_gather_t(table_hbm, idx_hbm, out_hbm, idx_v, rows_v, t_v, sem):
    wid = lax.axis_index("s") * NC + lax.axis_index("c")
    base = wid * BPW
    pltpu.sync_copy(idx_hbm.at[pl.ds(base, BPW)], idx_v)

    lane = lax.iota(jnp.int32, 16)

    for c in range(NCHUNK):
        pltpu.async_copy(
            table_hbm.at[idx_v.at[pl.ds(c * CHUNK, CHUNK)]],
            rows_v.at[:, pl.ds(0, D)],
            sem,
        ).wait()

        @plsc.parallel_loop(0, D, 1, unroll=2)
        def trow(e):
            col = jnp.full((16,), e, jnp.int32)
            for j in range(CHUNK // 16):
                v = plsc.load_gather(rows_v, [lane + 16 * j, col])
                t_v[e, pl.ds(c * CHUNK + 16 * j, 16)] = v

    pltpu.sync_copy(t_v, out_hbm.at[:, pl.ds(base, BPW)])


def kernel(task_index, task_index_to_mask):
    idx = task_index.reshape(B).astype(jnp.int32)
    return _sc_gather_t(task_index_to_mask, idx)[:, :, None]


# speedup vs baseline: 1.6546x; 1.1561x over previous
import functools

import jax
import jax.numpy as jnp
from jax import lax
from jax.experimental import pallas as pl
from jax.experimental.pallas import tpu as pltpu
from jax.experimental.pallas import tpu_sc as plsc

B = 16384
D = 128
NC = 2
NS = 16
NW = NC * NS
BPW = B // NW

_mesh = plsc.VectorSubcoreMesh(core_axis_name="c", subcore_axis_name="s")


@functools.partial(
    pl.kernel,
    mesh=_mesh,
    out_type=jax.ShapeDtypeStruct((B, D), jnp.float32),
    compiler_params=pltpu.CompilerParams(needs_layout_passes=False),
    scratch_types=[
        pltpu.VMEM((BPW,), jnp.int32),
        pltpu.VMEM((BPW, D), jnp.float32),
        pltpu.SemaphoreType.DMA,
    ],
)
def _sc_gather(table_hbm, idx_hbm, out_hbm, idx_v, rows_v, sem):
    wid = lax.axis_index("s") * NC + lax.axis_index("c")
    base = wid * BPW
    pltpu.sync_copy(idx_hbm.at[pl.ds(base, BPW)], idx_v)
    pltpu.async_copy(table_hbm.at[idx_v], rows_v, sem).wait()
    pltpu.sync_copy(rows_v, out_hbm.at[pl.ds(base, BPW)])


def _tt_body(x_ref, o_ref):
    o_ref[...] = jnp.transpose(x_ref[...], (1, 0))[None, :, :]


_tc_transpose = pl.pallas_call(
    _tt_body,
    grid=(NW,),
    in_specs=[pl.BlockSpec((BPW, D), lambda i: (i, 0))],
    out_specs=pl.BlockSpec((1, D, BPW), lambda i: (0, 0, i)),
    out_shape=jax.ShapeDtypeStruct((1, D, B), jnp.float32),
)


def kernel(task_index, task_index_to_mask):
    idx = task_index.reshape(B).astype(jnp.int32)
    rows = _sc_gather(task_index_to_mask, idx)
    return jnp.transpose(_tc_transpose(rows), (1, 2, 0))


# speedup vs baseline: 1.7142x; 1.0360x over previous
import functools

import jax
import jax.numpy as jnp
from jax import lax
from jax.experimental import pallas as pl
from jax.experimental.pallas import tpu as pltpu
from jax.experimental.pallas import tpu_sc as plsc

B = 16384
D = 128
NC = 2
NS = 16
NW = NC * NS
BPW = B // NW
CHUNK = 128
NCHUNK = BPW // CHUNK

_mesh = plsc.VectorSubcoreMesh(core_axis_name="c", subcore_axis_name="s")

_GDN = lax.GatherDimensionNumbers(
    offset_dims=(), collapsed_slice_dims=(0,), start_index_map=(0,)
)


def _perm(v, idx):
    return lax.gather(
        v,
        idx[:, None],
        dimension_numbers=_GDN,
        slice_sizes=(1,),
        mode=lax.GatherScatterMode.PROMISE_IN_BOUNDS,
    )


def _transpose16(vs, lane):
    for s in (1, 2, 4, 8):
        msk = (lane & s) == 0
        x = lane ^ s
        new = list(vs)
        for i in range(16):
            if i & s:
                continue
            j = i + s
            a, b = vs[i], vs[j]
            new[i] = jnp.where(msk, a, _perm(b, x))
            new[j] = jnp.where(msk, _perm(a, x), b)
        vs = new
    return vs


@functools.partial(
    pl.kernel,
    mesh=_mesh,
    out_type=jax.ShapeDtypeStruct((D, B), jnp.float32),
    compiler_params=pltpu.CompilerParams(needs_layout_passes=False),
    scratch_types=[
        pltpu.VMEM((BPW,), jnp.int32),
        pltpu.VMEM((CHUNK, D), jnp.float32),
        pltpu.VMEM((D, BPW), jnp.float32),
        pltpu.SemaphoreType.DMA,
    ],
)
def _sc_gather_t(table_hbm, idx_hbm, out_hbm, idx_v, rows_v, t_v, sem):
    wid = lax.axis_index("s") * NC + lax.axis_index("c")
    base = wid * BPW
    pltpu.sync_copy(idx_hbm.at[pl.ds(base, BPW)], idx_v)

    lane = lax.iota(jnp.int32, 16)

    def do_chunk(c, _):
        pltpu.async_copy(
            table_hbm.at[idx_v.at[pl.ds(c * CHUNK, CHUNK)]], rows_v, sem
        ).wait()

        @plsc.parallel_loop(0, CHUNK // 16, 1)
        def bi_loop(bi):
            r0 = bi * 16
            for bj in range(D // 16):
                vs = [rows_v[r0 + r, pl.ds(bj * 16, 16)] for r in range(16)]
                ws = _transpose16(vs, lane)
                for r in range(16):
                    t_v[bj * 16 + r, pl.ds(c * CHUNK + r0, 16)] = ws[r]

        return 0

    lax.fori_loop(0, NCHUNK, do_chunk, 0)
    pltpu.sync_copy(t_v, out_hbm.at[:, pl.ds(base, BPW)])


def kernel(task_index, task_index_to_mask):
    idx = task_index.reshape(B).astype(jnp.int32)
    return _sc_gather_t(task_index_to_mask, idx)[:, :, None]


# speedup vs baseline: 2.2570x; 1.3167x over previous
import functools

import jax
import jax.numpy as jnp
from jax import lax
from jax.experimental import pallas as pl
from jax.experimental.pallas import tpu as pltpu
from jax.experimental.pallas import tpu_sc as plsc

B = 16384
D = 128
NC = 2
NS = 16
NW = NC * NS
BPW = B // NW
CHUNK = 128
NCHUNK = BPW // CHUNK

_mesh = plsc.VectorSubcoreMesh(core_axis_name="c", subcore_axis_name="s")

_GDN = lax.GatherDimensionNumbers(
    offset_dims=(), collapsed_slice_dims=(0,), start_index_map=(0,)
)


def _perm(v, idx):
    return lax.gather(
        v,
        idx[:, None],
        dimension_numbers=_GDN,
        slice_sizes=(1,),
        mode=lax.GatherScatterMode.PROMISE_IN_BOUNDS,
    )


def _transpose16(vs, lane):
    for s in (1, 2, 4, 8):
        msk = (lane & s) == 0
        x = lane ^ s
        new = list(vs)
        for i in range(16):
            if i & s:
                continue
            j = i + s
            a, b = vs[i], vs[j]
            new[i] = jnp.where(msk, a, _perm(b, x))
            new[j] = jnp.where(msk, _perm(a, x), b)
        vs = new
    return vs


@functools.partial(
    pl.kernel,
    mesh=_mesh,
    out_type=jax.ShapeDtypeStruct((D, B), jnp.float32),
    compiler_params=pltpu.CompilerParams(needs_layout_passes=False),
    scratch_types=[
        pltpu.VMEM((BPW,), jnp.int32),
        pltpu.VMEM((CHUNK, D), jnp.float32),
        pltpu.VMEM((CHUNK, D), jnp.float32),
        pltpu.VMEM((D, BPW), jnp.float32),
        pltpu.SemaphoreType.DMA,
        pltpu.SemaphoreType.DMA,
    ],
)
def _sc_gather_t(table_hbm, idx_hbm, out_hbm, idx_v, rows_a, rows_b, t_v,
                 sem_a, sem_b):
    wid = lax.axis_index("s") * NC + lax.axis_index("c")
    base = wid * BPW
    pltpu.sync_copy(idx_hbm.at[pl.ds(base, BPW)], idx_v)

    lane = lax.iota(jnp.int32, 16)

    def transpose_chunk(rows_v, c):
        @plsc.parallel_loop(0, (CHUNK // 16) * (D // 16), 1)
        def bb_loop(bb):
            bi = bb // (D // 16)
            bj = bb % (D // 16)
            r0 = bi * 16
            vs = [rows_v[r0 + r, pl.ds(bj * 16, 16)] for r in range(16)]
            ws = _transpose16(vs, lane)
            for r in range(16):
                t_v[bj * 16 + r, pl.ds(c * CHUNK + r0, 16)] = ws[r]

    def gather_chunk(c, rows_v, sem):
        pltpu.async_copy(
            table_hbm.at[idx_v.at[pl.ds(c * CHUNK, CHUNK)]], rows_v, sem
        )

    gather_chunk(0, rows_a, sem_a)

    def do_pair(cc, _):
        c0 = cc * 2
        gather_chunk(c0 + 1, rows_b, sem_b)
        pltpu.make_async_copy(
            table_hbm.at[idx_v.at[pl.ds(0, CHUNK)]], rows_a, sem_a
        ).wait()
        transpose_chunk(rows_a, c0)

        @pl.when(cc + 1 < NCHUNK // 2)
        def _():
            gather_chunk(c0 + 2, rows_a, sem_a)

        pltpu.make_async_copy(
            table_hbm.at[idx_v.at[pl.ds(0, CHUNK)]], rows_b, sem_b
        ).wait()
        transpose_chunk(rows_b, c0 + 1)
        return 0

    lax.fori_loop(0, NCHUNK // 2, do_pair, 0)
    pltpu.sync_copy(t_v, out_hbm.at[:, pl.ds(base, BPW)])


def kernel(task_index, task_index_to_mask):
    idx = task_index.reshape(B).astype(jnp.int32)
    return _sc_gather_t(task_index_to_mask, idx)[:, :, None]
